# baseline (speedup 1.0000x reference)
import numpy as np

import jax
import jax.numpy as jnp
from jax import lax
from jax.experimental import pallas as pl
from jax.experimental.pallas import tpu as pltpu
from jax.experimental.pallas import tpu_sc as plsc

_K = 32
_L = 16
_B, _N, _S = 4, 16384, 1024
_NC, _NS = 2, 16
_NW = _NC * _NS
_QPW = (_B * _S) // _NW
_WPB = _NW // _B
_NV = _N // _L

def _lanes():
    return lax.iota(jnp.int32, _L)


def _gather(x, idx):
    return x.at[idx].get(mode="promise_in_bounds")


def _b32(pred):
    return jnp.where(pred, jnp.int32(1), jnp.int32(0))


def _lex_less(da, ia, db, ib):
    return _b32(da < db) | (_b32(da == db) & _b32(ia < ib))


def _min_tree(v):
    lanes = _lanes()
    for s in (8, 4, 2, 1):
        v = jnp.minimum(v, _gather(v, jnp.bitwise_xor(lanes, s)))
    return v


def _ce(d, i, s, k):
    lanes = _lanes()
    pidx = jnp.bitwise_xor(lanes, s)
    bs = (lanes >> int(np.log2(s))) & 1
    bk = (lanes >> int(np.log2(k))) & 1
    up = jnp.bitwise_xor(jnp.bitwise_xor(bs, bk), 1)
    pd = _gather(d, pidx)
    pi = _gather(i, pidx)
    cond = _lex_less(d, i, pd, pi) == up
    return jnp.where(cond, d, pd), jnp.where(cond, i, pi)


def _sortnet16(d, i):
    for k in (2, 4, 8, 16):
        s = k // 2
        while s >= 1:
            d, i = _ce(d, i, s, k)
            s //= 2
    return d, i


def _cleanup(d, i):
    for s in (8, 4, 2, 1):
        d, i = _ce(d, i, s, 16)
    return d, i


def _merge16(cd, ci, b0d, b0i, b1d, b1i):
    rcd, rci = jnp.flip(cd), jnp.flip(ci)
    t = _lex_less(b1d, b1i, rcd, rci) > 0
    wd = jnp.where(t, b1d, rcd)
    wi = jnp.where(t, b1i, rci)
    wd, wi = _cleanup(wd, wi)
    rwd, rwi = jnp.flip(wd), jnp.flip(wi)
    t2 = _lex_less(b0d, b0i, rwd, rwi) > 0
    lod = jnp.where(t2, b0d, rwd)
    loi = jnp.where(t2, b0i, rwi)
    hid = jnp.where(t2, rwd, b0d)
    hii = jnp.where(t2, rwi, b0i)
    lod, loi = _cleanup(lod, loi)
    hid, hii = _cleanup(hid, hii)
    return lod, loi, hid, hii


def _c15():
    return jnp.full((_L,), _L - 1, jnp.int32)


def _bf16r(v):
    bits = lax.bitcast_convert_type(v, jnp.int32)
    r = bits + jnp.int32(0x7FFF) + ((bits >> 16) & 1)
    return lax.bitcast_convert_type(r & jnp.int32(-65536), jnp.float32)


def _shift_insert(bd, bi, cdv, civ):
    lanes = _lanes()
    lm1 = jnp.maximum(lanes - 1, 0)
    c15 = _c15()
    keep = _lex_less(bd, bi, cdv, civ)
    ksh = _gather(keep, lm1)
    ksh = jnp.where(lanes == 0, jnp.full((_L,), 1, jnp.int32), ksh)
    candpos = ((1 - keep) & ksh) > 0
    keepb = keep > 0
    shd = _gather(bd, lm1)
    shi = _gather(bi, lm1)
    nbd = jnp.where(keepb, bd, jnp.where(candpos, cdv, shd))
    nbi = jnp.where(keepb, bi, jnp.where(candpos, civ, shi))
    b15d = _gather(bd, c15)
    b15i = _gather(bi, c15)
    t = _lex_less(b15d, b15i, cdv, civ)
    m = jnp.int32(0) - t
    spd_bits = (lax.bitcast_convert_type(cdv, jnp.int32) & m) | (
        lax.bitcast_convert_type(b15d, jnp.int32) & ~m)
    spd = lax.bitcast_convert_type(spd_bits, jnp.float32)
    spi = (civ & m) | (b15i & ~m)
    return nbd, nbi, spd, spi


def _insert_one(cdv, civ, b0d, b0i, b1d, b1i):
    nb0d, nb0i, c1d, c1i = _shift_insert(b0d, b0i, cdv, civ)
    nb1d, nb1i, _, _ = _shift_insert(b1d, b1i, c1d, c1i)
    return nb0d, nb0i, nb1d, nb1i


def _make_sc_body(qpw):
  def _sc_body(xyz_hbm, q_hbm, outd_hbm, outi_hbm,
               xyz_v, xn_v, q_v, dloc, iloc, bd_r, bi_r):
    wid = lax.axis_index("s") * _NC + lax.axis_index("c")
    b = wid // _WPB
    base = wid * qpw

    pltpu.sync_copy(xyz_hbm.at[b], xyz_v)
    pltpu.sync_copy(q_hbm.at[wid], q_v)

    def xn_body(v, _):
        xs = xyz_v[0, pl.ds(v * _L, _L)]
        ys = xyz_v[1, pl.ds(v * _L, _L)]
        zs = xyz_v[2, pl.ds(v * _L, _L)]
        xn_v[pl.ds(v * _L, _L)] = (xs * xs + ys * ys) + zs * zs
        xyz_v[0, pl.ds(v * _L, _L)] = _bf16r(xs)
        xyz_v[1, pl.ds(v * _L, _L)] = _bf16r(ys)
        xyz_v[2, pl.ds(v * _L, _L)] = _bf16r(zs)
        return 0

    lax.fori_loop(0, _NV, xn_body, 0)

    inf = jnp.float32(jnp.inf)
    lanes = _lanes()

    def make_slow(slot, d, kth_s, off):
        def slow():
            b0d = bd_r[slot, pl.ds(0, _L)]
            b0i = bi_r[slot, pl.ds(0, _L)]
            b1d = bd_r[slot, pl.ds(_L, _L)]
            b1i = bi_r[slot, pl.ds(_L, _L)]
            ivec = lanes + off
            qm = _b32(d < jnp.full((_L,), kth_s))
            dm = jnp.where(qm > 0, d, inf)
            cdv = dm
            civ = ivec
            for s in (8, 4, 2, 1):
                pdd = _gather(cdv, jnp.bitwise_xor(lanes, s))
                pii = _gather(civ, jnp.bitwise_xor(lanes, s))
                tk = _lex_less(cdv, civ, pdd, pii) > 0
                cdv = jnp.where(tk, cdv, pdd)
                civ = jnp.where(tk, civ, pii)
            rem = qm & _b32(ivec != civ)
            mt2 = _min_tree(jnp.where(rem > 0, d, inf))
            multi = mt2[0] < kth_s

            def do_single():
                nb0d, nb0i, nb1d, nb1i = _insert_one(
                    cdv, civ, b0d, b0i, b1d, b1i)
                bd_r[slot, pl.ds(0, _L)] = nb0d
                bi_r[slot, pl.ds(0, _L)] = nb0i
                bd_r[slot, pl.ds(_L, _L)] = nb1d
                bi_r[slot, pl.ds(_L, _L)] = nb1i
                return nb1d[_L - 1]

            def do_multi():
                sd, si = _sortnet16(d, ivec)
                nb0d, nb0i, nb1d, nb1i = _merge16(
                    sd, si, b0d, b0i, b1d, b1i)
                bd_r[slot, pl.ds(0, _L)] = nb0d
                bi_r[slot, pl.ds(0, _L)] = nb0i
                bd_r[slot, pl.ds(_L, _L)] = nb1d
                bi_r[slot, pl.ds(_L, _L)] = nb1i
                return nb1d[_L - 1]

            return lax.cond(multi, do_multi, do_single)

        return slow

    _NSLOT = 4

    def per_group(jg, _):
        qx, qy, qz, qn = [], [], [], []
        for slot in range(_NSLOT):
            j = jg * _NSLOT + slot
            row = (j // _L) * _L
            lane = jnp.full((_L,), j % _L, jnp.int32)
            qxv = _gather(q_v[0, pl.ds(row, _L)], lane)
            qyv = _gather(q_v[1, pl.ds(row, _L)], lane)
            qzv = _gather(q_v[2, pl.ds(row, _L)], lane)
            qn.append((qxv * qxv + qyv * qyv) + qzv * qzv)
            qx.append(_bf16r(qxv))
            qy.append(_bf16r(qyv))
            qz.append(_bf16r(qzv))
            bd_r[slot, pl.ds(0, _L)] = jnp.full((_L,), inf)
            bi_r[slot, pl.ds(0, _L)] = jnp.zeros((_L,), jnp.int32)
            bd_r[slot, pl.ds(_L, _L)] = jnp.full((_L,), inf)
            bi_r[slot, pl.ds(_L, _L)] = jnp.zeros((_L,), jnp.int32)

        def dist_at(slot, off):
            xs = xyz_v[0, pl.ds(off, _L)]
            ys = xyz_v[1, pl.ds(off, _L)]
            zs = xyz_v[2, pl.ds(off, _L)]
            xn = xn_v[pl.ds(off, _L)]
            dot = (xs * qx[slot] + ys * qy[slot]) + zs * qz[slot]
            return (-2.0 * dot + qn[slot]) + xn

        _G = 2

        def rescan_group(slot, kth0, goff):
            def ubody(u, k):
                off = goff + u * _L
                d = dist_at(slot, off)
                mu = _min_tree(d)[0]
                return lax.cond(
                    mu < k, make_slow(slot, d, k, off), lambda k=k: k)

            return lax.fori_loop(0, _G, ubody, kth0)

        def scan_body(g, kths):
            goff = g * (_G * _L)
            pts = []
            for u in range(_G):
                off = goff + u * _L
                pts.append((
                    xyz_v[0, pl.ds(off, _L)],
                    xyz_v[1, pl.ds(off, _L)],
                    xyz_v[2, pl.ds(off, _L)],
                    xn_v[pl.ds(off, _L)],
                ))
            mts = []
            for slot in range(_NSLOT):
                gmin = None
                for xs, ys, zs, xn in pts:
                    dot = (xs * qx[slot] + ys * qy[slot]) + zs * qz[slot]
                    d = (-2.0 * dot + qn[slot]) + xn
                    gmin = d if gmin is None else jnp.minimum(gmin, d)
                mts.append(_min_tree(gmin)[0])
            hits = [mts[s] < kths[s] for s in range(_NSLOT)]
            any_hit = hits[0]
            for s in range(1, _NSLOT):
                any_hit = any_hit | hits[s]

            def slow_all():
                out = []
                for slot in range(_NSLOT):
                    nk = lax.cond(
                        hits[slot],
                        lambda slot=slot, k=kths[slot]: rescan_group(
                            slot, k, goff),
                        lambda k=kths[slot]: k,
                    )
                    out.append(nk)
                return tuple(out)

            return lax.cond(any_hit, slow_all, lambda: kths)

        lax.fori_loop(0, _NV // _G, scan_body, (inf,) * _NSLOT)

        for slot in range(_NSLOT):
            j = jg * _NSLOT + slot
            dloc[j, pl.ds(0, _L)] = bd_r[slot, pl.ds(0, _L)]
            dloc[j, pl.ds(_L, _L)] = bd_r[slot, pl.ds(_L, _L)]
            iloc[j, pl.ds(0, _L)] = bi_r[slot, pl.ds(0, _L)]
            iloc[j, pl.ds(_L, _L)] = bi_r[slot, pl.ds(_L, _L)]
        return 0

    lax.fori_loop(0, qpw // _NSLOT, per_group, 0)

    pltpu.sync_copy(dloc, outd_hbm.at[pl.ds(base, qpw)])
    pltpu.sync_copy(iloc, outi_hbm.at[pl.ds(base, qpw)])

  return _sc_body


def _knn_sc(xyz, new_xyz):
    s_sc = new_xyz.shape[1]
    qpw = (_B * s_sc) // _NW
    xyz_soa = xyz.transpose(0, 2, 1)
    q_soa = (
        new_xyz.reshape(_B, _WPB, qpw, 3)
        .transpose(0, 1, 3, 2)
        .reshape(_NW, 3, qpw)
    )
    kfun = pl.kernel(
        _make_sc_body(qpw),
        out_type=[
            jax.ShapeDtypeStruct((_B * s_sc, _K), jnp.float32),
            jax.ShapeDtypeStruct((_B * s_sc, _K), jnp.int32),
        ],
        mesh=plsc.VectorSubcoreMesh(core_axis_name="c", subcore_axis_name="s"),
        scratch_types=[
            pltpu.VMEM((3, _N), jnp.float32),
            pltpu.VMEM((_N,), jnp.float32),
            pltpu.VMEM((3, qpw), jnp.float32),
            pltpu.VMEM((qpw, _K), jnp.float32),
            pltpu.VMEM((qpw, _K), jnp.int32),
            pltpu.VMEM((4, 2 * _L), jnp.float32),
            pltpu.VMEM((4, 2 * _L), jnp.int32),
        ],
    )
    outd, outi = kfun(xyz_soa, q_soa)
    return outd.reshape(_B, s_sc, _K), outi.reshape(_B, s_sc, _K)


_SBLK = 128


def _tc_body(qT_ref, xT_ref, dists_ref, idx_ref, dist_scratch):
    q = qT_ref[0]
    x = xT_ref[0]
    n = x.shape[-1]
    dot = jax.lax.dot_general(
        q, x, (((0,), (0,)), ((), ())), preferred_element_type=jnp.float32
    )
    qn = jnp.sum(q * q, axis=0)[:, None]
    xn = jnp.sum(x * x, axis=0)[None, :]
    dist_scratch[...] = (-2.0 * dot + qn) + xn
    col = jax.lax.broadcasted_iota(jnp.int32, (_SBLK, n), 1)

    def body(k, _):
        d = dist_scratch[...]
        m = jnp.min(d, axis=1, keepdims=True)
        amin = jnp.min(
            jnp.where(d == m, col, jnp.int32(n)), axis=1, keepdims=True
        )
        dists_ref[0, k, :] = m[:, 0]
        idx_ref[0, k, :] = amin[:, 0]
        dist_scratch[...] = jnp.where(col == amin, jnp.float32(jnp.inf), d)
        return 0

    jax.lax.fori_loop(0, _K, body, 0)


def _knn_tc(xyz, new_xyz):
    b, n, _ = xyz.shape
    s = new_xyz.shape[1]
    pad = jnp.zeros((b, 5, n), jnp.float32)
    xT = jnp.concatenate([xyz.transpose(0, 2, 1), pad], axis=1)
    qpad = jnp.zeros((b, 5, s), jnp.float32)
    qT = jnp.concatenate([new_xyz.transpose(0, 2, 1), qpad], axis=1)

    grid = (b, s // _SBLK)
    dists_t, idx_t = pl.pallas_call(
        _tc_body,
        grid=grid,
        in_specs=[
            pl.BlockSpec((1, 8, _SBLK), lambda bi, si: (bi, 0, si)),
            pl.BlockSpec((1, 8, n), lambda bi, si: (bi, 0, 0)),
        ],
        out_specs=[
            pl.BlockSpec((1, _K, _SBLK), lambda bi, si: (bi, 0, si)),
            pl.BlockSpec((1, _K, _SBLK), lambda bi, si: (bi, 0, si)),
        ],
        out_shape=[
            jax.ShapeDtypeStruct((b, _K, s), jnp.float32),
            jax.ShapeDtypeStruct((b, _K, s), jnp.int32),
        ],
        scratch_shapes=[pltpu.VMEM((_SBLK, n), jnp.float32)],
    )(qT, xT)
    return dists_t.transpose(0, 2, 1), idx_t.transpose(0, 2, 1)


_S_SC = 512


def kernel(nsample, xyz, new_xyz):
    del nsample
    sc_d, sc_i = _knn_sc(xyz, new_xyz[:, :_S_SC])
    tc_d, tc_i = _knn_tc(xyz, new_xyz[:, _S_SC:])
    return (
        jnp.concatenate([sc_d, tc_d], axis=1),
        jnp.concatenate([sc_i, tc_i], axis=1),
    )


# speedup vs baseline: 1.1646x; 1.1646x over previous
import numpy as np

import jax
import jax.numpy as jnp
from jax import lax
from jax.experimental import pallas as pl
from jax.experimental.pallas import tpu as pltpu
from jax.experimental.pallas import tpu_sc as plsc

_K = 32
_L = 16
_B, _N, _S = 4, 16384, 1024
_NC, _NS = 2, 16
_NW = _NC * _NS
_QPW = (_B * _S) // _NW
_WPB = _NW // _B
_NV = _N // _L

def _lanes():
    return lax.iota(jnp.int32, _L)


def _gather(x, idx):
    return x.at[idx].get(mode="promise_in_bounds")


def _b32(pred):
    return jnp.where(pred, jnp.int32(1), jnp.int32(0))


def _lex_less(da, ia, db, ib):
    return _b32(da < db) | (_b32(da == db) & _b32(ia < ib))


def _min_tree(v):
    lanes = _lanes()
    for s in (8, 4, 2, 1):
        v = jnp.minimum(v, _gather(v, jnp.bitwise_xor(lanes, s)))
    return v


def _ce(d, i, s, k):
    lanes = _lanes()
    pidx = jnp.bitwise_xor(lanes, s)
    bs = (lanes >> int(np.log2(s))) & 1
    bk = (lanes >> int(np.log2(k))) & 1
    up = jnp.bitwise_xor(jnp.bitwise_xor(bs, bk), 1)
    pd = _gather(d, pidx)
    pi = _gather(i, pidx)
    cond = _lex_less(d, i, pd, pi) == up
    return jnp.where(cond, d, pd), jnp.where(cond, i, pi)


def _sortnet16(d, i):
    for k in (2, 4, 8, 16):
        s = k // 2
        while s >= 1:
            d, i = _ce(d, i, s, k)
            s //= 2
    return d, i


def _cleanup(d, i):
    for s in (8, 4, 2, 1):
        d, i = _ce(d, i, s, 16)
    return d, i


def _merge16(cd, ci, b0d, b0i, b1d, b1i):
    rcd, rci = jnp.flip(cd), jnp.flip(ci)
    t = _lex_less(b1d, b1i, rcd, rci) > 0
    wd = jnp.where(t, b1d, rcd)
    wi = jnp.where(t, b1i, rci)
    wd, wi = _cleanup(wd, wi)
    rwd, rwi = jnp.flip(wd), jnp.flip(wi)
    t2 = _lex_less(b0d, b0i, rwd, rwi) > 0
    lod = jnp.where(t2, b0d, rwd)
    loi = jnp.where(t2, b0i, rwi)
    hid = jnp.where(t2, rwd, b0d)
    hii = jnp.where(t2, rwi, b0i)
    lod, loi = _cleanup(lod, loi)
    hid, hii = _cleanup(hid, hii)
    return lod, loi, hid, hii


def _c15():
    return jnp.full((_L,), _L - 1, jnp.int32)


def _bf16r(v):
    bits = lax.bitcast_convert_type(v, jnp.int32)
    r = bits + jnp.int32(0x7FFF) + ((bits >> 16) & 1)
    return lax.bitcast_convert_type(r & jnp.int32(-65536), jnp.float32)


def _shift_insert(bd, bi, cdv, civ):
    lanes = _lanes()
    lm1 = jnp.maximum(lanes - 1, 0)
    c15 = _c15()
    keep = _lex_less(bd, bi, cdv, civ)
    ksh = _gather(keep, lm1)
    ksh = jnp.where(lanes == 0, jnp.full((_L,), 1, jnp.int32), ksh)
    candpos = ((1 - keep) & ksh) > 0
    keepb = keep > 0
    shd = _gather(bd, lm1)
    shi = _gather(bi, lm1)
    nbd = jnp.where(keepb, bd, jnp.where(candpos, cdv, shd))
    nbi = jnp.where(keepb, bi, jnp.where(candpos, civ, shi))
    b15d = _gather(bd, c15)
    b15i = _gather(bi, c15)
    t = _lex_less(b15d, b15i, cdv, civ)
    m = jnp.int32(0) - t
    spd_bits = (lax.bitcast_convert_type(cdv, jnp.int32) & m) | (
        lax.bitcast_convert_type(b15d, jnp.int32) & ~m)
    spd = lax.bitcast_convert_type(spd_bits, jnp.float32)
    spi = (civ & m) | (b15i & ~m)
    return nbd, nbi, spd, spi


def _insert_one(cdv, civ, b0d, b0i, b1d, b1i):
    nb0d, nb0i, c1d, c1i = _shift_insert(b0d, b0i, cdv, civ)
    nb1d, nb1i, _, _ = _shift_insert(b1d, b1i, c1d, c1i)
    return nb0d, nb0i, nb1d, nb1i


def _make_sc_body(qpw):
  def _sc_body(xyz_hbm, q_hbm, outd_hbm, outi_hbm,
               xyz_v, xn_v, q_v, dloc, iloc, bd_r, bi_r):
    wid = lax.axis_index("s") * _NC + lax.axis_index("c")
    b = wid // _WPB
    base = wid * qpw

    pltpu.sync_copy(xyz_hbm.at[b], xyz_v)
    pltpu.sync_copy(q_hbm.at[wid], q_v)

    def xn_body(v, _):
        xs = xyz_v[0, pl.ds(v * _L, _L)]
        ys = xyz_v[1, pl.ds(v * _L, _L)]
        zs = xyz_v[2, pl.ds(v * _L, _L)]
        xn_v[pl.ds(v * _L, _L)] = (xs * xs + ys * ys) + zs * zs
        xyz_v[0, pl.ds(v * _L, _L)] = _bf16r(xs)
        xyz_v[1, pl.ds(v * _L, _L)] = _bf16r(ys)
        xyz_v[2, pl.ds(v * _L, _L)] = _bf16r(zs)
        return 0

    lax.fori_loop(0, _NV, xn_body, 0)

    inf = jnp.float32(jnp.inf)
    lanes = _lanes()

    def make_slow(slot, d, kth_s, off):
        def slow():
            b0d = bd_r[slot, pl.ds(0, _L)]
            b0i = bi_r[slot, pl.ds(0, _L)]
            b1d = bd_r[slot, pl.ds(_L, _L)]
            b1i = bi_r[slot, pl.ds(_L, _L)]
            ivec = lanes + off
            qm = _b32(d < jnp.full((_L,), kth_s))
            dm = jnp.where(qm > 0, d, inf)
            cdv = dm
            civ = ivec
            for s in (8, 4, 2, 1):
                pdd = _gather(cdv, jnp.bitwise_xor(lanes, s))
                pii = _gather(civ, jnp.bitwise_xor(lanes, s))
                tk = _lex_less(cdv, civ, pdd, pii) > 0
                cdv = jnp.where(tk, cdv, pdd)
                civ = jnp.where(tk, civ, pii)
            rem = qm & _b32(ivec != civ)
            mt2 = _min_tree(jnp.where(rem > 0, d, inf))
            multi = mt2[0] < kth_s

            def do_single():
                nb0d, nb0i, nb1d, nb1i = _insert_one(
                    cdv, civ, b0d, b0i, b1d, b1i)
                bd_r[slot, pl.ds(0, _L)] = nb0d
                bi_r[slot, pl.ds(0, _L)] = nb0i
                bd_r[slot, pl.ds(_L, _L)] = nb1d
                bi_r[slot, pl.ds(_L, _L)] = nb1i
                return nb1d[_L - 1]

            def do_multi():
                sd, si = _sortnet16(d, ivec)
                nb0d, nb0i, nb1d, nb1i = _merge16(
                    sd, si, b0d, b0i, b1d, b1i)
                bd_r[slot, pl.ds(0, _L)] = nb0d
                bi_r[slot, pl.ds(0, _L)] = nb0i
                bd_r[slot, pl.ds(_L, _L)] = nb1d
                bi_r[slot, pl.ds(_L, _L)] = nb1i
                return nb1d[_L - 1]

            return lax.cond(multi, do_multi, do_single)

        return slow

    _NSLOT = 4

    def per_group(jg, _):
        qx, qy, qz, qn = [], [], [], []
        for slot in range(_NSLOT):
            j = jg * _NSLOT + slot
            row = (j // _L) * _L
            lane = jnp.full((_L,), j % _L, jnp.int32)
            qxv = _gather(q_v[0, pl.ds(row, _L)], lane)
            qyv = _gather(q_v[1, pl.ds(row, _L)], lane)
            qzv = _gather(q_v[2, pl.ds(row, _L)], lane)
            qn.append((qxv * qxv + qyv * qyv) + qzv * qzv)
            qx.append(_bf16r(qxv))
            qy.append(_bf16r(qyv))
            qz.append(_bf16r(qzv))
            bd_r[slot, pl.ds(0, _L)] = jnp.full((_L,), inf)
            bi_r[slot, pl.ds(0, _L)] = jnp.zeros((_L,), jnp.int32)
            bd_r[slot, pl.ds(_L, _L)] = jnp.full((_L,), inf)
            bi_r[slot, pl.ds(_L, _L)] = jnp.zeros((_L,), jnp.int32)

        def scan_body(v, kths):
            off = v * _L
            xs = xyz_v[0, pl.ds(off, _L)]
            ys = xyz_v[1, pl.ds(off, _L)]
            zs = xyz_v[2, pl.ds(off, _L)]
            xn = xn_v[pl.ds(off, _L)]
            ds = []
            mts = []
            for slot in range(_NSLOT):
                dot = (xs * qx[slot] + ys * qy[slot]) + zs * qz[slot]
                d = (-2.0 * dot + qn[slot]) + xn
                ds.append(d)
                mts.append(_min_tree(d)[0])
            hits = [mts[s] < kths[s] for s in range(_NSLOT)]
            any_hit = hits[0]
            for s in range(1, _NSLOT):
                any_hit = any_hit | hits[s]

            def slow_all():
                out = []
                for slot in range(_NSLOT):
                    nk = lax.cond(
                        hits[slot],
                        make_slow(slot, ds[slot], kths[slot], off),
                        lambda k=kths[slot]: k,
                    )
                    out.append(nk)
                return tuple(out)

            return lax.cond(any_hit, slow_all, lambda: kths)

        lax.fori_loop(0, _NV, scan_body, (inf,) * _NSLOT)

        for slot in range(_NSLOT):
            j = jg * _NSLOT + slot
            dloc[j, pl.ds(0, _L)] = bd_r[slot, pl.ds(0, _L)]
            dloc[j, pl.ds(_L, _L)] = bd_r[slot, pl.ds(_L, _L)]
            iloc[j, pl.ds(0, _L)] = bi_r[slot, pl.ds(0, _L)]
            iloc[j, pl.ds(_L, _L)] = bi_r[slot, pl.ds(_L, _L)]
        return 0

    lax.fori_loop(0, qpw // _NSLOT, per_group, 0)

    pltpu.sync_copy(dloc, outd_hbm.at[pl.ds(base, qpw)])
    pltpu.sync_copy(iloc, outi_hbm.at[pl.ds(base, qpw)])

  return _sc_body


def _knn_sc(xyz, new_xyz):
    s_sc = new_xyz.shape[1]
    qpw = (_B * s_sc) // _NW
    xyz_soa = xyz.transpose(0, 2, 1)
    q_soa = (
        new_xyz.reshape(_B, _WPB, qpw, 3)
        .transpose(0, 1, 3, 2)
        .reshape(_NW, 3, qpw)
    )
    kfun = pl.kernel(
        _make_sc_body(qpw),
        out_type=[
            jax.ShapeDtypeStruct((_B * s_sc, _K), jnp.float32),
            jax.ShapeDtypeStruct((_B * s_sc, _K), jnp.int32),
        ],
        mesh=plsc.VectorSubcoreMesh(core_axis_name="c", subcore_axis_name="s"),
        scratch_types=[
            pltpu.VMEM((3, _N), jnp.float32),
            pltpu.VMEM((_N,), jnp.float32),
            pltpu.VMEM((3, qpw), jnp.float32),
            pltpu.VMEM((qpw, _K), jnp.float32),
            pltpu.VMEM((qpw, _K), jnp.int32),
            pltpu.VMEM((4, 2 * _L), jnp.float32),
            pltpu.VMEM((4, 2 * _L), jnp.int32),
        ],
    )
    outd, outi = kfun(xyz_soa, q_soa)
    return outd.reshape(_B, s_sc, _K), outi.reshape(_B, s_sc, _K)


_SBLK = 128


def _tc_body(qT_ref, xT_ref, dists_ref, idx_ref, dist_scratch):
    q = qT_ref[0]
    x = xT_ref[0]
    n = x.shape[-1]
    dot = jax.lax.dot_general(
        q, x, (((0,), (0,)), ((), ())), preferred_element_type=jnp.float32
    )
    qn = jnp.sum(q * q, axis=0)[:, None]
    xn = jnp.sum(x * x, axis=0)[None, :]
    dist_scratch[...] = (-2.0 * dot + qn) + xn
    col = jax.lax.broadcasted_iota(jnp.int32, (_SBLK, n), 1)

    def body(k, _):
        d = dist_scratch[...]
        m = jnp.min(d, axis=1, keepdims=True)
        amin = jnp.min(
            jnp.where(d == m, col, jnp.int32(n)), axis=1, keepdims=True
        )
        dists_ref[0, k, :] = m[:, 0]
        idx_ref[0, k, :] = amin[:, 0]
        dist_scratch[...] = jnp.where(col == amin, jnp.float32(jnp.inf), d)
        return 0

    jax.lax.fori_loop(0, _K, body, 0)


def _knn_tc(xyz, new_xyz):
    b, n, _ = xyz.shape
    s = new_xyz.shape[1]
    pad = jnp.zeros((b, 5, n), jnp.float32)
    xT = jnp.concatenate([xyz.transpose(0, 2, 1), pad], axis=1)
    qpad = jnp.zeros((b, 5, s), jnp.float32)
    qT = jnp.concatenate([new_xyz.transpose(0, 2, 1), qpad], axis=1)

    grid = (b, s // _SBLK)
    dists_t, idx_t = pl.pallas_call(
        _tc_body,
        grid=grid,
        in_specs=[
            pl.BlockSpec((1, 8, _SBLK), lambda bi, si: (bi, 0, si)),
            pl.BlockSpec((1, 8, n), lambda bi, si: (bi, 0, 0)),
        ],
        out_specs=[
            pl.BlockSpec((1, _K, _SBLK), lambda bi, si: (bi, 0, si)),
            pl.BlockSpec((1, _K, _SBLK), lambda bi, si: (bi, 0, si)),
        ],
        out_shape=[
            jax.ShapeDtypeStruct((b, _K, s), jnp.float32),
            jax.ShapeDtypeStruct((b, _K, s), jnp.int32),
        ],
        scratch_shapes=[pltpu.VMEM((_SBLK, n), jnp.float32)],
    )(qT, xT)
    return dists_t.transpose(0, 2, 1), idx_t.transpose(0, 2, 1)


_S_SC = 512


def kernel(nsample, xyz, new_xyz):
    del nsample
    sc_d, sc_i = _knn_sc(xyz, new_xyz[:, :_S_SC])
    tc_d, tc_i = _knn_tc(xyz, new_xyz[:, _S_SC:])
    return (
        jnp.concatenate([sc_d, tc_d], axis=1),
        jnp.concatenate([sc_i, tc_i], axis=1),
    )


# speedup vs baseline: 1.1799x; 1.0132x over previous
import numpy as np

import jax
import jax.numpy as jnp
from jax import lax
from jax.experimental import pallas as pl
from jax.experimental.pallas import tpu as pltpu
from jax.experimental.pallas import tpu_sc as plsc

_K = 32
_L = 16
_B, _N, _S = 4, 16384, 1024
_NC, _NS = 2, 16
_NW = _NC * _NS
_QPW = (_B * _S) // _NW
_WPB = _NW // _B
_NV = _N // _L

def _lanes():
    return lax.iota(jnp.int32, _L)


def _gather(x, idx):
    return x.at[idx].get(mode="promise_in_bounds")


def _b32(pred):
    return jnp.where(pred, jnp.int32(1), jnp.int32(0))


def _lex_less(da, ia, db, ib):
    return _b32(da < db) | (_b32(da == db) & _b32(ia < ib))


def _min_tree(v):
    lanes = _lanes()
    for s in (8, 4, 2, 1):
        v = jnp.minimum(v, _gather(v, jnp.bitwise_xor(lanes, s)))
    return v


def _ce(d, i, s, k):
    lanes = _lanes()
    pidx = jnp.bitwise_xor(lanes, s)
    bs = (lanes >> int(np.log2(s))) & 1
    bk = (lanes >> int(np.log2(k))) & 1
    up = jnp.bitwise_xor(jnp.bitwise_xor(bs, bk), 1)
    pd = _gather(d, pidx)
    pi = _gather(i, pidx)
    cond = _lex_less(d, i, pd, pi) == up
    return jnp.where(cond, d, pd), jnp.where(cond, i, pi)


def _sortnet16(d, i):
    for k in (2, 4, 8, 16):
        s = k // 2
        while s >= 1:
            d, i = _ce(d, i, s, k)
            s //= 2
    return d, i


def _cleanup(d, i):
    for s in (8, 4, 2, 1):
        d, i = _ce(d, i, s, 16)
    return d, i


def _merge16(cd, ci, b0d, b0i, b1d, b1i):
    rcd, rci = jnp.flip(cd), jnp.flip(ci)
    t = _lex_less(b1d, b1i, rcd, rci) > 0
    wd = jnp.where(t, b1d, rcd)
    wi = jnp.where(t, b1i, rci)
    wd, wi = _cleanup(wd, wi)
    rwd, rwi = jnp.flip(wd), jnp.flip(wi)
    t2 = _lex_less(b0d, b0i, rwd, rwi) > 0
    lod = jnp.where(t2, b0d, rwd)
    loi = jnp.where(t2, b0i, rwi)
    hid = jnp.where(t2, rwd, b0d)
    hii = jnp.where(t2, rwi, b0i)
    lod, loi = _cleanup(lod, loi)
    hid, hii = _cleanup(hid, hii)
    return lod, loi, hid, hii


def _c15():
    return jnp.full((_L,), _L - 1, jnp.int32)


def _bf16r(v):
    bits = lax.bitcast_convert_type(v, jnp.int32)
    r = bits + jnp.int32(0x7FFF) + ((bits >> 16) & 1)
    return lax.bitcast_convert_type(r & jnp.int32(-65536), jnp.float32)


def _shift_insert(bd, bi, cdv, civ):
    lanes = _lanes()
    lm1 = jnp.maximum(lanes - 1, 0)
    c15 = _c15()
    keep = _lex_less(bd, bi, cdv, civ)
    ksh = _gather(keep, lm1)
    ksh = jnp.where(lanes == 0, jnp.full((_L,), 1, jnp.int32), ksh)
    candpos = ((1 - keep) & ksh) > 0
    keepb = keep > 0
    shd = _gather(bd, lm1)
    shi = _gather(bi, lm1)
    nbd = jnp.where(keepb, bd, jnp.where(candpos, cdv, shd))
    nbi = jnp.where(keepb, bi, jnp.where(candpos, civ, shi))
    b15d = _gather(bd, c15)
    b15i = _gather(bi, c15)
    t = _lex_less(b15d, b15i, cdv, civ)
    m = jnp.int32(0) - t
    spd_bits = (lax.bitcast_convert_type(cdv, jnp.int32) & m) | (
        lax.bitcast_convert_type(b15d, jnp.int32) & ~m)
    spd = lax.bitcast_convert_type(spd_bits, jnp.float32)
    spi = (civ & m) | (b15i & ~m)
    return nbd, nbi, spd, spi


def _insert_one(cdv, civ, b0d, b0i, b1d, b1i):
    nb0d, nb0i, c1d, c1i = _shift_insert(b0d, b0i, cdv, civ)
    nb1d, nb1i, _, _ = _shift_insert(b1d, b1i, c1d, c1i)
    return nb0d, nb0i, nb1d, nb1i


def _make_sc_body(qpw):
  def _sc_body(xyz_hbm, q_hbm, outd_hbm, outi_hbm,
               xyz_v, xn_v, q_v, dloc, iloc, bd_r, bi_r):
    wid = lax.axis_index("s") * _NC + lax.axis_index("c")
    b = wid // _WPB
    base = wid * qpw

    pltpu.sync_copy(xyz_hbm.at[b], xyz_v)
    pltpu.sync_copy(q_hbm.at[wid], q_v)

    def xn_body(v, _):
        xs = xyz_v[0, pl.ds(v * _L, _L)]
        ys = xyz_v[1, pl.ds(v * _L, _L)]
        zs = xyz_v[2, pl.ds(v * _L, _L)]
        xn_v[pl.ds(v * _L, _L)] = (xs * xs + ys * ys) + zs * zs
        xyz_v[0, pl.ds(v * _L, _L)] = -2.0 * _bf16r(xs)
        xyz_v[1, pl.ds(v * _L, _L)] = -2.0 * _bf16r(ys)
        xyz_v[2, pl.ds(v * _L, _L)] = -2.0 * _bf16r(zs)
        return 0

    lax.fori_loop(0, _NV, xn_body, 0)

    inf = jnp.float32(jnp.inf)
    lanes = _lanes()

    def make_slow(slot, d, kth_s, off):
        def slow():
            b0d = bd_r[slot, pl.ds(0, _L)]
            b0i = bi_r[slot, pl.ds(0, _L)]
            b1d = bd_r[slot, pl.ds(_L, _L)]
            b1i = bi_r[slot, pl.ds(_L, _L)]
            ivec = lanes + off
            qm = _b32(d < jnp.full((_L,), kth_s))
            dm = jnp.where(qm > 0, d, inf)
            cdv = dm
            civ = ivec
            for s in (8, 4, 2, 1):
                pdd = _gather(cdv, jnp.bitwise_xor(lanes, s))
                pii = _gather(civ, jnp.bitwise_xor(lanes, s))
                tk = _lex_less(cdv, civ, pdd, pii) > 0
                cdv = jnp.where(tk, cdv, pdd)
                civ = jnp.where(tk, civ, pii)
            rem = qm & _b32(ivec != civ)
            mt2 = _min_tree(jnp.where(rem > 0, d, inf))
            multi = mt2[0] < kth_s

            def do_single():
                nb0d, nb0i, nb1d, nb1i = _insert_one(
                    cdv, civ, b0d, b0i, b1d, b1i)
                bd_r[slot, pl.ds(0, _L)] = nb0d
                bi_r[slot, pl.ds(0, _L)] = nb0i
                bd_r[slot, pl.ds(_L, _L)] = nb1d
                bi_r[slot, pl.ds(_L, _L)] = nb1i
                return nb1d[_L - 1]

            def do_multi():
                sd, si = _sortnet16(d, ivec)
                nb0d, nb0i, nb1d, nb1i = _merge16(
                    sd, si, b0d, b0i, b1d, b1i)
                bd_r[slot, pl.ds(0, _L)] = nb0d
                bi_r[slot, pl.ds(0, _L)] = nb0i
                bd_r[slot, pl.ds(_L, _L)] = nb1d
                bi_r[slot, pl.ds(_L, _L)] = nb1i
                return nb1d[_L - 1]

            return lax.cond(multi, do_multi, do_single)

        return slow

    _NSLOT = 4

    def per_group(jg, _):
        qx, qy, qz, qn = [], [], [], []
        for slot in range(_NSLOT):
            j = jg * _NSLOT + slot
            row = (j // _L) * _L
            lane = jnp.full((_L,), j % _L, jnp.int32)
            qxv = _gather(q_v[0, pl.ds(row, _L)], lane)
            qyv = _gather(q_v[1, pl.ds(row, _L)], lane)
            qzv = _gather(q_v[2, pl.ds(row, _L)], lane)
            qn.append((qxv * qxv + qyv * qyv) + qzv * qzv)
            qx.append(_bf16r(qxv))
            qy.append(_bf16r(qyv))
            qz.append(_bf16r(qzv))
            bd_r[slot, pl.ds(0, _L)] = jnp.full((_L,), inf)
            bi_r[slot, pl.ds(0, _L)] = jnp.zeros((_L,), jnp.int32)
            bd_r[slot, pl.ds(_L, _L)] = jnp.full((_L,), inf)
            bi_r[slot, pl.ds(_L, _L)] = jnp.zeros((_L,), jnp.int32)

        def scan_body(v, kths):
            off = v * _L
            xs = xyz_v[0, pl.ds(off, _L)]
            ys = xyz_v[1, pl.ds(off, _L)]
            zs = xyz_v[2, pl.ds(off, _L)]
            xn = xn_v[pl.ds(off, _L)]
            ds = []
            mts = []
            for slot in range(_NSLOT):
                dot = (xs * qx[slot] + ys * qy[slot]) + zs * qz[slot]
                d = (dot + qn[slot]) + xn
                ds.append(d)
                mts.append(_min_tree(d)[0])
            hits = [mts[s] < kths[s] for s in range(_NSLOT)]
            any_hit = hits[0]
            for s in range(1, _NSLOT):
                any_hit = any_hit | hits[s]

            def slow_all():
                out = []
                for slot in range(_NSLOT):
                    nk = lax.cond(
                        hits[slot],
                        make_slow(slot, ds[slot], kths[slot], off),
                        lambda k=kths[slot]: k,
                    )
                    out.append(nk)
                return tuple(out)

            return lax.cond(any_hit, slow_all, lambda: kths)

        lax.fori_loop(0, _NV, scan_body, (inf,) * _NSLOT)

        for slot in range(_NSLOT):
            j = jg * _NSLOT + slot
            dloc[j, pl.ds(0, _L)] = bd_r[slot, pl.ds(0, _L)]
            dloc[j, pl.ds(_L, _L)] = bd_r[slot, pl.ds(_L, _L)]
            iloc[j, pl.ds(0, _L)] = bi_r[slot, pl.ds(0, _L)]
            iloc[j, pl.ds(_L, _L)] = bi_r[slot, pl.ds(_L, _L)]
        return 0

    lax.fori_loop(0, qpw // _NSLOT, per_group, 0)

    pltpu.sync_copy(dloc, outd_hbm.at[pl.ds(base, qpw)])
    pltpu.sync_copy(iloc, outi_hbm.at[pl.ds(base, qpw)])

  return _sc_body


def _knn_sc(xyz, new_xyz):
    s_sc = new_xyz.shape[1]
    qpw = (_B * s_sc) // _NW
    xyz_soa = xyz.transpose(0, 2, 1)
    q_soa = (
        new_xyz.reshape(_B, _WPB, qpw, 3)
        .transpose(0, 1, 3, 2)
        .reshape(_NW, 3, qpw)
    )
    kfun = pl.kernel(
        _make_sc_body(qpw),
        out_type=[
            jax.ShapeDtypeStruct((_B * s_sc, _K), jnp.float32),
            jax.ShapeDtypeStruct((_B * s_sc, _K), jnp.int32),
        ],
        mesh=plsc.VectorSubcoreMesh(core_axis_name="c", subcore_axis_name="s"),
        scratch_types=[
            pltpu.VMEM((3, _N), jnp.float32),
            pltpu.VMEM((_N,), jnp.float32),
            pltpu.VMEM((3, qpw), jnp.float32),
            pltpu.VMEM((qpw, _K), jnp.float32),
            pltpu.VMEM((qpw, _K), jnp.int32),
            pltpu.VMEM((4, 2 * _L), jnp.float32),
            pltpu.VMEM((4, 2 * _L), jnp.int32),
        ],
    )
    outd, outi = kfun(xyz_soa, q_soa)
    return outd.reshape(_B, s_sc, _K), outi.reshape(_B, s_sc, _K)


_SBLK = 256


def _tc_body(qT_ref, xT_ref, dists_ref, idx_ref, dist_scratch):
    q = qT_ref[0]
    x = xT_ref[0]
    n = x.shape[-1]
    dot = jax.lax.dot_general(
        q, x, (((0,), (0,)), ((), ())), preferred_element_type=jnp.float32
    )
    qn = jnp.sum(q * q, axis=0)[:, None]
    xn = jnp.sum(x * x, axis=0)[None, :]
    dist_scratch[...] = (-2.0 * dot + qn) + xn
    col = jax.lax.broadcasted_iota(jnp.int32, (_SBLK, n), 1)

    def body(k, _):
        d = dist_scratch[...]
        m = jnp.min(d, axis=1, keepdims=True)
        amin = jnp.min(
            jnp.where(d == m, col, jnp.int32(n)), axis=1, keepdims=True
        )
        dists_ref[0, k, :] = m[:, 0]
        idx_ref[0, k, :] = amin[:, 0]
        dist_scratch[...] = jnp.where(col == amin, jnp.float32(jnp.inf), d)
        return 0

    jax.lax.fori_loop(0, _K, body, 0)


def _knn_tc(xyz, new_xyz):
    b, n, _ = xyz.shape
    s = new_xyz.shape[1]
    pad = jnp.zeros((b, 5, n), jnp.float32)
    xT = jnp.concatenate([xyz.transpose(0, 2, 1), pad], axis=1)
    qpad = jnp.zeros((b, 5, s), jnp.float32)
    qT = jnp.concatenate([new_xyz.transpose(0, 2, 1), qpad], axis=1)

    grid = (b, s // _SBLK)
    dists_t, idx_t = pl.pallas_call(
        _tc_body,
        grid=grid,
        in_specs=[
            pl.BlockSpec((1, 8, _SBLK), lambda bi, si: (bi, 0, si)),
            pl.BlockSpec((1, 8, n), lambda bi, si: (bi, 0, 0)),
        ],
        out_specs=[
            pl.BlockSpec((1, _K, _SBLK), lambda bi, si: (bi, 0, si)),
            pl.BlockSpec((1, _K, _SBLK), lambda bi, si: (bi, 0, si)),
        ],
        out_shape=[
            jax.ShapeDtypeStruct((b, _K, s), jnp.float32),
            jax.ShapeDtypeStruct((b, _K, s), jnp.int32),
        ],
        scratch_shapes=[pltpu.VMEM((_SBLK, n), jnp.float32)],
    )(qT, xT)
    return dists_t.transpose(0, 2, 1), idx_t.transpose(0, 2, 1)


_S_SC = 512


def kernel(nsample, xyz, new_xyz):
    del nsample
    sc_d, sc_i = _knn_sc(xyz, new_xyz[:, :_S_SC])
    tc_d, tc_i = _knn_tc(xyz, new_xyz[:, _S_SC:])
    return (
        jnp.concatenate([sc_d, tc_d], axis=1),
        jnp.concatenate([sc_i, tc_i], axis=1),
    )
